# baseline (speedup 1.0000x reference)
import functools

import jax
import jax.numpy as jnp
from jax import lax
from jax.experimental import pallas as pl
from jax.experimental.pallas import tpu as pltpu
from jax.experimental.pallas import tpu_sc as plsc

_NL = 3
_NA = 3
_NC = 80
_BS = 16
_NT = 500
_KPA = 512
_KL = _NA * _KPA
_STRIDES = (8.0, 16.0, 32.0)
_GRIDS = ((80, 80), (40, 40), (20, 20))
_ROWS = tuple(_BS * _NA * ny * nx for (ny, nx) in _GRIDS)
_OWNER = (0, 1, 1)
_NSUB = 16
_CPT = _KL // _NSUB
_NVR = _CPT // 16
_GW = 96
_OCH = tuple(r // 32 for r in _ROWS)
_OPT = tuple(-(-c // 128) for c in _OCH)
_ORS = tuple(32 * p for p in _OPT)


_CPW = _KL // 32


def _candidate_rows(i, kv, bb_v, bi_v, anc_v, iota):
    ny, nx = _GRIDS[i]
    a = kv >> 9
    t = kv & 511
    tm = t < _NT
    tc = jnp.minimum(t, _NT - 1)
    t4 = tc * 4
    bx = plsc.load_gather(bb_v, [t4])
    by = plsc.load_gather(bb_v, [t4 + 1])
    bw = plsc.load_gather(bb_v, [t4 + 2])
    bh = plsc.load_gather(bb_v, [t4 + 3])
    b = plsc.load_gather(bi_v, [tc])
    a2 = (i * _NA + a) * 2
    aw = plsc.load_gather(anc_v, [a2])
    ah = plsc.load_gather(anc_v, [a2 + 1])
    rw = bw * (nx * _STRIDES[i]) / aw
    rh = bh * (ny * _STRIDES[i]) / ah
    m = (rw < 4.0) & (rw > 0.25) & (rh < 4.0) & (rh > 0.25) & tm
    gx = bx * nx
    gy = by * ny
    gi = jnp.clip(gx.astype(jnp.int32), 0, nx - 1)
    gj = jnp.clip(gy.astype(jnp.int32), 0, ny - 1)
    row = ((b * _NA + a) * ny + gj) * nx + gi
    return row, m


def _sc_layer(i, tflat, anchors, bboxes, batch_idx):
    mesh = plsc.VectorSubcoreMesh(core_axis_name="c", subcore_axis_name="s")
    out_type = (
        jax.ShapeDtypeStruct((_KL * _GW,), jnp.float32),
        jax.ShapeDtypeStruct((_KL,), jnp.int32),
    )
    scratch = [
        pltpu.VMEM((_NT * 4,), jnp.float32),
        pltpu.VMEM((_NT,), jnp.int32),
        pltpu.VMEM((_NL * _NA * 2,), jnp.float32),
        pltpu.VMEM((_CPW,), jnp.int32),
        pltpu.VMEM((_CPT,), jnp.int32),
        pltpu.VMEM((_CPT,), jnp.int32),
        pltpu.VMEM((_CPT,), jnp.int32),
        pltpu.VMEM((_CPW * _GW,), jnp.int32),
        pltpu.VMEM((_CPW * _GW,), jnp.float32),
        pltpu.VMEM_SHARED((_ROWS[i] + 16,), jnp.int32),
    ]
    stride = _SSTR[i] * 128

    @functools.partial(
        pl.kernel, out_type=out_type, mesh=mesh, scratch_types=scratch,
        compiler_params=pltpu.CompilerParams(needs_layout_passes=False,
                                             use_tc_tiling_on_sc=False))
    def k(p_h, anc_h, bb_h, bi_h, ps_h, got_h,
          bb_v, bi_v, anc_v, row_v, scat_v, val_v, got_v, eidx_v, rows_v,
          dbuf):
        cid = lax.axis_index("c")
        sid = lax.axis_index("s")
        wid = sid * 2 + cid
        pltpu.sync_copy(bb_h, bb_v)
        pltpu.sync_copy(bi_h, bi_v)
        pltpu.sync_copy(anc_h, anc_v)
        iota = lax.iota(jnp.int32, 16)
        trash = _ROWS[i]

        @pl.when(cid == 0)
        def _():
            for j in range(_NVR):
                kv = sid * _CPT + j * 16 + iota
                row, m = _candidate_rows(i, kv, bb_v, bi_v, anc_v, iota)
                sl = pl.ds(j * 16, 16)
                scat_v[sl] = jnp.where(m, row, trash)
                val_v[sl] = kv
            pltpu.sync_copy(val_v, dbuf.at[scat_v])

        for j in range(_CPW // 16):
            kv = wid * _CPW + j * 16 + iota
            row, _ = _candidate_rows(i, kv, bb_v, bi_v, anc_v, iota)
            row_v[pl.ds(j * 16, 16)] = row

        def qbody(q, _):
            rb = plsc.load_gather(row_v, [jnp.full((16,), q, jnp.int32)])
            for j in range(_GW // 16):
                c = jnp.minimum(j * 16 + iota, 84)
                eidx_v[pl.ds(q * _GW + j * 16, 16)] = rb + c * stride
            return 0

        lax.fori_loop(0, _CPW, qbody, 0)
        pltpu.sync_copy(p_h.at[eidx_v], rows_v)
        pltpu.sync_copy(rows_v, ps_h.at[pl.ds(wid * _CPW * _GW,
                                              _CPW * _GW)])

        plsc.subcore_barrier()

        @pl.when(cid == 0)
        def _():
            pltpu.sync_copy(dbuf.at[scat_v], got_v)
            pltpu.sync_copy(got_v, got_h.at[pl.ds(sid * _CPT, _CPT)])

    return k(tflat, anchors, bboxes, batch_idx)


_SSTR = (2400, 608, 160)


def _transflat(pf, br, ostr):
    rows = pf.shape[0]
    ns = br // 128
    whole = br == rows
    ons = ostr if whole else ns

    def body(p_ref, o_ref):
        t = p_ref[...].T
        if whole and ostr * 128 > rows:
            t = jnp.pad(t, ((0, 0), (0, ostr * 128 - rows)))
        o_ref[...] = t.reshape(85, ons, 128)

    return pl.pallas_call(
        body,
        grid=(rows // br,),
        in_specs=[pl.BlockSpec((br, 85), lambda g: (g, 0))],
        out_specs=pl.BlockSpec((85, ons, 128), lambda g: (0, g, 0)),
        out_shape=jax.ShapeDtypeStruct((85, ostr, 128), jnp.float32),
    )(pf)


def _atan_pos(x):
    inv = x > 1.0
    t = jnp.where(inv, 1.0 / jnp.maximum(x, 1e-30), x)
    t2 = t * t
    p = -0.0117212
    p = p * t2 + 0.05265332
    p = p * t2 - 0.11643287
    p = p * t2 + 0.19354346
    p = p * t2 - 0.33262347
    p = p * t2 + 0.99997726
    r = t * p
    return jnp.where(inv, (jnp.pi / 2.0) - r, r)


def _final_losses(pbs, xcs, gots, objs, bm, bwh_c, cls_c, anchors):
    def body(pb0, pb1, pb2, xc0, xc1, xc2, g0, g1, g2,
             ob0, ob1, ob2, bm_ref, bwhc_ref, clsc_ref, anc_ref,
             lbox_o, lcls_o, tot_o):
        objr = (ob0, ob1, ob2)
        pb = (pb0, pb1, pb2)
        xc = (xc0, xc1, xc2)
        got = (g0, g1, g2)
        r12 = lax.broadcasted_iota(jnp.int32, (12, 128), 0)
        c12 = lax.broadcasted_iota(jnp.int32, (12, 128), 1)
        k12 = r12 * 128 + c12
        a12 = r12 >> 2
        tm12 = (k12 & 511) < _NT
        kcol = lax.broadcasted_iota(jnp.int32, (_KL, 1), 0)
        acol = kcol >> 9
        tmcol = (kcol & 511) < _NT
        bx = bm_ref[0]
        by = bm_ref[1]
        bw = bm_ref[2]
        bh = bm_ref[3]
        bwc = bwhc_ref[:, 0:1]
        bhc = bwhc_ref[:, 1:2]
        clsc = clsc_ref[:, 0:1]
        l_box = 0.0
        l_cls = 0.0
        l_obj = 0.0
        eps = 1e-7
        for i in range(_NL):
            ny, nx = _GRIDS[i]
            sgain = _STRIDES[i]

            def asel(av, c):
                s0 = anc_ref[i, 0, c]
                s1 = anc_ref[i, 1, c]
                s2 = anc_ref[i, 2, c]
                return jnp.where(av == 0, s0, jnp.where(av == 1, s1, s2))

            aw12 = asel(a12, 0)
            ah12 = asel(a12, 1)
            awc = asel(acol, 0)
            ahc = asel(acol, 1)
            rw = bw * (nx * sgain) / aw12
            rh = bh * (ny * sgain) / ah12
            m12 = (rw < 4.0) & (rw > 0.25) & (rh < 4.0) & (rh > 0.25) & tm12
            rwc = bwc * (nx * sgain) / awc
            rhc = bhc * (ny * sgain) / ahc
            mcol = ((rwc < 4.0) & (rwc > 0.25) & (rhc < 4.0) & (rhc > 0.25)
                    & tmcol)
            n = jnp.sum(jnp.where(m12, 1.0, 0.0))
            safe_n = jnp.maximum(n, 1.0)

            px = pb[i][0]
            py = pb[i][1]
            pw = pb[i][2]
            ph = pb[i][3]
            pobj = pb[i][4]
            sx = 1.0 / (1.0 + jnp.exp(-px))
            sy = 1.0 / (1.0 + jnp.exp(-py))
            sw = 1.0 / (1.0 + jnp.exp(-pw))
            sh = 1.0 / (1.0 + jnp.exp(-ph))
            pxy_x = sx * 2.0 - 0.5
            pxy_y = sy * 2.0 - 0.5
            pw_ = (sw * 2.0) ** 2 * aw12
            ph_ = (sh * 2.0) ** 2 * ah12
            gx = bx * nx
            gy = by * ny
            tbx = gx - gx.astype(jnp.int32).astype(jnp.float32)
            tby = gy - gy.astype(jnp.int32).astype(jnp.float32)
            tbw = bw * nx
            tbh = bh * ny
            b1x1 = pxy_x - pw_ * 0.5
            b1x2 = pxy_x + pw_ * 0.5
            b1y1 = pxy_y - ph_ * 0.5
            b1y2 = pxy_y + ph_ * 0.5
            b2x1 = tbx - tbw * 0.5
            b2x2 = tbx + tbw * 0.5
            b2y1 = tby - tbh * 0.5
            b2y2 = tby + tbh * 0.5
            iw = jnp.maximum(jnp.minimum(b1x2, b2x2)
                             - jnp.maximum(b1x1, b2x1), 0.0)
            ih = jnp.maximum(jnp.minimum(b1y2, b2y2)
                             - jnp.maximum(b1y1, b2y1), 0.0)
            inter = iw * ih
            union = pw_ * ph_ + tbw * tbh - inter + eps
            iou = inter / union
            cw = jnp.maximum(b1x2, b2x2) - jnp.minimum(b1x1, b2x1)
            ch = jnp.maximum(b1y2, b2y2) - jnp.minimum(b1y1, b2y1)
            c2 = cw * cw + ch * ch + eps
            rho2 = ((b2x1 + b2x2 - b1x1 - b1x2) ** 2
                    + (b2y1 + b2y2 - b1y1 - b1y2) ** 2) * 0.25
            dv = (_atan_pos(tbw / (tbh + eps)) - _atan_pos(pw_ / (ph_ + eps)))
            v = (4.0 / (jnp.pi ** 2)) * dv * dv
            alpha = v / (v - iou + (1.0 + eps))
            ciou = iou - (rho2 / c2 + v * alpha)
            box_sum = jnp.sum(jnp.where(m12, 1.0 - ciou, 0.0))
            l_box = l_box + jnp.where(n > 0.0, box_sum / safe_n, 0.0)

            winner = m12 & (got[i][:] == k12)
            tsum = jnp.sum(jnp.where(winner, pobj * jnp.maximum(ciou, 0.0),
                                     0.0))
            xo = objr[i][0]
            sp = (jnp.maximum(xo, 0.0)
                  + jnp.log(1.0 + jnp.exp(-jnp.abs(xo))))
            if _SSTR[i] * 128 > _ROWS[i]:
                ro = lax.broadcasted_iota(jnp.int32, xo.shape, 0)
                sp = jnp.where(ro >= _ROWS[i] // 128, 0.0, sp)
            sp_sum = jnp.sum(sp)
            l_obj = l_obj + (sp_sum - tsum) / float(_ROWS[i])

            xcls = xc[i][:]
            oh = lax.broadcasted_iota(jnp.int32, (_KL, _NC), 1) == clsc
            elem = (jnp.maximum(xcls, 0.0) - jnp.where(oh, xcls, 0.0)
                    + jnp.log(1.0 + jnp.exp(-jnp.abs(xcls))))
            cls_sum = jnp.sum(jnp.where(mcol, elem, 0.0))
            l_cls = l_cls + jnp.where(n > 0.0,
                                      cls_sum / (safe_n * float(_NC)), 0.0)

        def put(ref, v):
            ref[...] = jnp.full((1, 1), v, jnp.float32)

        put(lbox_o, l_box * (0.05 * _BS))
        put(lcls_o, l_cls * (0.5 * _BS))
        put(tot_o, (l_box * 0.05 + l_cls * 0.5 + l_obj) * _BS)

    smem = pl.BlockSpec(memory_space=pltpu.SMEM)
    out_shape = tuple(jax.ShapeDtypeStruct((1, 1), jnp.float32)
                      for _ in range(3))
    obj_specs = [pl.BlockSpec((1, s, 128), lambda g: (4, 0, 0))
                 for s in _SSTR]

    def full(a):
        nd = len(a.shape)
        return pl.BlockSpec(a.shape, lambda g, _n=nd: (0,) * _n)

    arrs = (*pbs, *xcs, *gots)
    tails = (bm, bwh_c, cls_c)
    return pl.pallas_call(
        body,
        grid=(1,),
        in_specs=([full(a) for a in arrs] + obj_specs
                  + [full(a) for a in tails] + [smem]),
        out_specs=[pl.BlockSpec((1, 1), lambda g: (0, 0))] * 3,
        out_shape=out_shape,
    )(*arrs, *objs, *tails, anchors)


def kernel(p0, p1, p2, anchors, bboxes, batch_idx, cls):
    p0f = p0.reshape(_ROWS[0], 85)
    p1f = p1.reshape(_ROWS[1], 85)
    p2f = p2.reshape(_ROWS[2], 85)
    batch_idx = batch_idx.astype(jnp.int32)
    cls = cls.astype(jnp.int32)

    anc_f = anchors.reshape(-1)
    bb_f = bboxes.reshape(-1)
    t2 = _transflat(p2f, 19200, _SSTR[2])
    ps2, got2 = _sc_layer(2, t2.reshape(-1), anc_f, bb_f, batch_idx)
    p1f, _ = lax.optimization_barrier((p1f, t2))
    t1 = _transflat(p1f, 15360, _SSTR[1])
    ps1, got1 = _sc_layer(1, t1.reshape(-1), anc_f, bb_f, batch_idx)
    p0f, _ = lax.optimization_barrier((p0f, t1))
    t0 = _transflat(p0f, 12288, _SSTR[0])
    ps0, got0 = _sc_layer(0, t0.reshape(-1), anc_f, bb_f, batch_idx)
    ps0, ps1, ps2 = (p.reshape(_KL, _GW) for p in (ps0, ps1, ps2))
    objs = [t0, t1, t2]

    def tile3(v):
        vp = jnp.pad(v, (0, _KPA - _NT))
        return jnp.tile(vp, 3)

    bm = jnp.stack([tile3(bboxes[:, c]).reshape(12, 128) for c in range(4)])
    bwh_c = jnp.stack([tile3(bboxes[:, 2]), tile3(bboxes[:, 3])], axis=1)
    cls_c = tile3(cls).reshape(_KL, 1)

    pbs, xcs, gots = [], [], []
    for ps, got in ((ps0, got0), (ps1, got1), (ps2, got2)):
        pbs.append(ps[:, :5].T.reshape(5, 12, 128))
        xcs.append(ps[:, 5:85])
        gots.append(got.reshape(12, 128))

    lb, lc, tot = _final_losses(pbs, xcs, gots, objs, bm, bwh_c, cls_c,
                                anchors)
    return (lb.reshape(1), lc.reshape(1), tot.reshape(1))


# speedup vs baseline: 1.1904x; 1.1904x over previous
import functools

import jax
import jax.numpy as jnp
from jax import lax
from jax.experimental import pallas as pl
from jax.experimental.pallas import tpu as pltpu
from jax.experimental.pallas import tpu_sc as plsc

_NL = 3
_NA = 3
_NC = 80
_BS = 16
_NT = 500
_KPA = 512
_KL = _NA * _KPA
_STRIDES = (8.0, 16.0, 32.0)
_GRIDS = ((80, 80), (40, 40), (20, 20))
_ROWS = tuple(_BS * _NA * ny * nx for (ny, nx) in _GRIDS)
_OWNER = (0, 1, 1)
_NSUB = 16
_CPT = _KL // _NSUB
_NVR = _CPT // 16
_GW = 96
_OCH = tuple(r // 32 for r in _ROWS)
_OPT = tuple(-(-c // 128) for c in _OCH)
_ORS = tuple(32 * p for p in _OPT)


_CPW = _KL // 32


def _candidate_rows(i, kv, bb_v, bi_v, anc_v, iota):
    ny, nx = _GRIDS[i]
    a = kv >> 9
    t = kv & 511
    tm = t < _NT
    tc = jnp.minimum(t, _NT - 1)
    t4 = tc * 4
    bx = plsc.load_gather(bb_v, [t4])
    by = plsc.load_gather(bb_v, [t4 + 1])
    bw = plsc.load_gather(bb_v, [t4 + 2])
    bh = plsc.load_gather(bb_v, [t4 + 3])
    b = plsc.load_gather(bi_v, [tc])
    a2 = (i * _NA + a) * 2
    aw = plsc.load_gather(anc_v, [a2])
    ah = plsc.load_gather(anc_v, [a2 + 1])
    rw = bw * (nx * _STRIDES[i]) / aw
    rh = bh * (ny * _STRIDES[i]) / ah
    m = (rw < 4.0) & (rw > 0.25) & (rh < 4.0) & (rh > 0.25) & tm
    gx = bx * nx
    gy = by * ny
    gi = jnp.clip(gx.astype(jnp.int32), 0, nx - 1)
    gj = jnp.clip(gy.astype(jnp.int32), 0, ny - 1)
    row = ((b * _NA + a) * ny + gj) * nx + gi
    return row, m


def _sc_layer(i, tflat, anchors, bboxes, batch_idx):
    mesh = plsc.VectorSubcoreMesh(core_axis_name="c", subcore_axis_name="s")
    out_type = (
        jax.ShapeDtypeStruct((_KL * _GW,), jnp.float32),
        jax.ShapeDtypeStruct((_KL,), jnp.int32),
    )
    scratch = [
        pltpu.VMEM((_NT * 4,), jnp.float32),
        pltpu.VMEM((_NT,), jnp.int32),
        pltpu.VMEM((_NL * _NA * 2,), jnp.float32),
        pltpu.VMEM((_CPW,), jnp.int32),
        pltpu.VMEM((_CPT,), jnp.int32),
        pltpu.VMEM((_CPT,), jnp.int32),
        pltpu.VMEM((_CPT,), jnp.int32),
        pltpu.VMEM((_CPW * _GW,), jnp.int32),
        pltpu.VMEM((_CPW * _GW,), jnp.float32),
        pltpu.VMEM_SHARED((_ROWS[i] + 16,), jnp.int32),
    ]
    stride = _SSTR[i] * 128

    @functools.partial(
        pl.kernel, out_type=out_type, mesh=mesh, scratch_types=scratch,
        compiler_params=pltpu.CompilerParams(needs_layout_passes=False,
                                             use_tc_tiling_on_sc=False))
    def k(p_h, anc_h, bb_h, bi_h, ps_h, got_h,
          bb_v, bi_v, anc_v, row_v, scat_v, val_v, got_v, eidx_v, rows_v,
          dbuf):
        cid = lax.axis_index("c")
        sid = lax.axis_index("s")
        wid = sid * 2 + cid
        pltpu.sync_copy(bb_h, bb_v)
        pltpu.sync_copy(bi_h, bi_v)
        pltpu.sync_copy(anc_h, anc_v)
        iota = lax.iota(jnp.int32, 16)
        trash = _ROWS[i]

        @pl.when(cid == 0)
        def _():
            for j in range(_NVR):
                kv = sid * _CPT + j * 16 + iota
                row, m = _candidate_rows(i, kv, bb_v, bi_v, anc_v, iota)
                sl = pl.ds(j * 16, 16)
                scat_v[sl] = jnp.where(m, row, trash)
                val_v[sl] = kv
            pltpu.sync_copy(val_v, dbuf.at[scat_v])

        for j in range(_CPW // 16):
            kv = wid * _CPW + j * 16 + iota
            row, _ = _candidate_rows(i, kv, bb_v, bi_v, anc_v, iota)
            row_v[pl.ds(j * 16, 16)] = row

        def qbody(q, _):
            rb = plsc.load_gather(row_v, [jnp.full((16,), q, jnp.int32)])
            for j in range(_GW // 16):
                c = jnp.minimum(j * 16 + iota, 84)
                eidx_v[pl.ds(q * _GW + j * 16, 16)] = rb + c * stride
            return 0

        lax.fori_loop(0, _CPW, qbody, 0)
        pltpu.sync_copy(p_h.at[eidx_v], rows_v)
        pltpu.sync_copy(rows_v, ps_h.at[pl.ds(wid * _CPW * _GW,
                                              _CPW * _GW)])

        plsc.subcore_barrier()

        @pl.when(cid == 0)
        def _():
            pltpu.sync_copy(dbuf.at[scat_v], got_v)
            pltpu.sync_copy(got_v, got_h.at[pl.ds(sid * _CPT, _CPT)])

    return k(tflat, anchors, bboxes, batch_idx)


_SSTR = (2400, 608, 160)


def _transflat(pf, br, ostr):
    rows = pf.shape[0]
    ns = br // 128
    whole = br == rows
    ons = ostr if whole else ns

    def body(p_ref, o_ref):
        t = p_ref[...].T
        if whole and ostr * 128 > rows:
            t = jnp.pad(t, ((0, 0), (0, ostr * 128 - rows)))
        o_ref[...] = t.reshape(85, ons, 128)

    return pl.pallas_call(
        body,
        grid=(rows // br,),
        in_specs=[pl.BlockSpec((br, 85), lambda g: (g, 0))],
        out_specs=pl.BlockSpec((85, ons, 128), lambda g: (0, g, 0)),
        out_shape=jax.ShapeDtypeStruct((85, ostr, 128), jnp.float32),
    )(pf)


def _atan_pos(x):
    inv = x > 1.0
    t = jnp.where(inv, 1.0 / jnp.maximum(x, 1e-30), x)
    t2 = t * t
    p = -0.0117212
    p = p * t2 + 0.05265332
    p = p * t2 - 0.11643287
    p = p * t2 + 0.19354346
    p = p * t2 - 0.33262347
    p = p * t2 + 0.99997726
    r = t * p
    return jnp.where(inv, (jnp.pi / 2.0) - r, r)


def _final_losses(pbs, xcs, gots, objs, bm, bwh_c, cls_c, anchors):
    def body(pb0, pb1, pb2, xc0, xc1, xc2, g0, g1, g2,
             ob0, ob1, ob2, bm_ref, bwhc_ref, clsc_ref, anc_ref,
             lbox_o, lcls_o, tot_o):
        objr = (ob0, ob1, ob2)
        pb = (pb0, pb1, pb2)
        xc = (xc0, xc1, xc2)
        got = (g0, g1, g2)
        r12 = lax.broadcasted_iota(jnp.int32, (12, 128), 0)
        c12 = lax.broadcasted_iota(jnp.int32, (12, 128), 1)
        k12 = r12 * 128 + c12
        a12 = r12 >> 2
        tm12 = (k12 & 511) < _NT
        kcol = lax.broadcasted_iota(jnp.int32, (_KL, 1), 0)
        acol = kcol >> 9
        tmcol = (kcol & 511) < _NT
        bx = bm_ref[0]
        by = bm_ref[1]
        bw = bm_ref[2]
        bh = bm_ref[3]
        bwc = bwhc_ref[:, 0:1]
        bhc = bwhc_ref[:, 1:2]
        clsc = clsc_ref[:, 0:1]
        l_box = 0.0
        l_cls = 0.0
        l_obj = 0.0
        eps = 1e-7
        for i in range(_NL):
            ny, nx = _GRIDS[i]
            sgain = _STRIDES[i]

            def asel(av, c):
                s0 = anc_ref[i, 0, c]
                s1 = anc_ref[i, 1, c]
                s2 = anc_ref[i, 2, c]
                return jnp.where(av == 0, s0, jnp.where(av == 1, s1, s2))

            aw12 = asel(a12, 0)
            ah12 = asel(a12, 1)
            awc = asel(acol, 0)
            ahc = asel(acol, 1)
            rw = bw * (nx * sgain) / aw12
            rh = bh * (ny * sgain) / ah12
            m12 = (rw < 4.0) & (rw > 0.25) & (rh < 4.0) & (rh > 0.25) & tm12
            rwc = bwc * (nx * sgain) / awc
            rhc = bhc * (ny * sgain) / ahc
            mcol = ((rwc < 4.0) & (rwc > 0.25) & (rhc < 4.0) & (rhc > 0.25)
                    & tmcol)
            n = jnp.sum(jnp.where(m12, 1.0, 0.0))
            safe_n = jnp.maximum(n, 1.0)

            px = pb[i][0]
            py = pb[i][1]
            pw = pb[i][2]
            ph = pb[i][3]
            pobj = pb[i][4]
            sx = 1.0 / (1.0 + jnp.exp(-px))
            sy = 1.0 / (1.0 + jnp.exp(-py))
            sw = 1.0 / (1.0 + jnp.exp(-pw))
            sh = 1.0 / (1.0 + jnp.exp(-ph))
            pxy_x = sx * 2.0 - 0.5
            pxy_y = sy * 2.0 - 0.5
            pw_ = (sw * 2.0) ** 2 * aw12
            ph_ = (sh * 2.0) ** 2 * ah12
            gx = bx * nx
            gy = by * ny
            tbx = gx - gx.astype(jnp.int32).astype(jnp.float32)
            tby = gy - gy.astype(jnp.int32).astype(jnp.float32)
            tbw = bw * nx
            tbh = bh * ny
            b1x1 = pxy_x - pw_ * 0.5
            b1x2 = pxy_x + pw_ * 0.5
            b1y1 = pxy_y - ph_ * 0.5
            b1y2 = pxy_y + ph_ * 0.5
            b2x1 = tbx - tbw * 0.5
            b2x2 = tbx + tbw * 0.5
            b2y1 = tby - tbh * 0.5
            b2y2 = tby + tbh * 0.5
            iw = jnp.maximum(jnp.minimum(b1x2, b2x2)
                             - jnp.maximum(b1x1, b2x1), 0.0)
            ih = jnp.maximum(jnp.minimum(b1y2, b2y2)
                             - jnp.maximum(b1y1, b2y1), 0.0)
            inter = iw * ih
            union = pw_ * ph_ + tbw * tbh - inter + eps
            iou = inter / union
            cw = jnp.maximum(b1x2, b2x2) - jnp.minimum(b1x1, b2x1)
            ch = jnp.maximum(b1y2, b2y2) - jnp.minimum(b1y1, b2y1)
            c2 = cw * cw + ch * ch + eps
            rho2 = ((b2x1 + b2x2 - b1x1 - b1x2) ** 2
                    + (b2y1 + b2y2 - b1y1 - b1y2) ** 2) * 0.25
            dv = (_atan_pos(tbw / (tbh + eps)) - _atan_pos(pw_ / (ph_ + eps)))
            v = (4.0 / (jnp.pi ** 2)) * dv * dv
            alpha = v / (v - iou + (1.0 + eps))
            ciou = iou - (rho2 / c2 + v * alpha)
            box_sum = jnp.sum(jnp.where(m12, 1.0 - ciou, 0.0))
            l_box = l_box + jnp.where(n > 0.0, box_sum / safe_n, 0.0)

            winner = m12 & (got[i][:] == k12)
            tsum = jnp.sum(jnp.where(winner, pobj * jnp.maximum(ciou, 0.0),
                                     0.0))
            xo = objr[i][0]
            sp = (jnp.maximum(xo, 0.0)
                  + jnp.log(1.0 + jnp.exp(-jnp.abs(xo))))
            if _SSTR[i] * 128 > _ROWS[i]:
                ro = lax.broadcasted_iota(jnp.int32, xo.shape, 0)
                sp = jnp.where(ro >= _ROWS[i] // 128, 0.0, sp)
            sp_sum = jnp.sum(sp)
            l_obj = l_obj + (sp_sum - tsum) / float(_ROWS[i])

            xcls = xc[i][:]
            oh = lax.broadcasted_iota(jnp.int32, (_KL, _NC), 1) == clsc
            elem = (jnp.maximum(xcls, 0.0) - jnp.where(oh, xcls, 0.0)
                    + jnp.log(1.0 + jnp.exp(-jnp.abs(xcls))))
            cls_sum = jnp.sum(jnp.where(mcol, elem, 0.0))
            l_cls = l_cls + jnp.where(n > 0.0,
                                      cls_sum / (safe_n * float(_NC)), 0.0)

        def put(ref, v):
            ref[...] = jnp.full((1, 1), v, jnp.float32)

        put(lbox_o, l_box * (0.05 * _BS))
        put(lcls_o, l_cls * (0.5 * _BS))
        put(tot_o, (l_box * 0.05 + l_cls * 0.5 + l_obj) * _BS)

    smem = pl.BlockSpec(memory_space=pltpu.SMEM)
    out_shape = tuple(jax.ShapeDtypeStruct((1, 1), jnp.float32)
                      for _ in range(3))
    obj_specs = [pl.BlockSpec((1, s, 128), lambda g: (4, 0, 0))
                 for s in _SSTR]

    def full(a):
        nd = len(a.shape)
        return pl.BlockSpec(a.shape, lambda g, _n=nd: (0,) * _n)

    arrs = (*pbs, *xcs, *gots)
    tails = (bm, bwh_c, cls_c)
    return pl.pallas_call(
        body,
        grid=(1,),
        in_specs=([full(a) for a in arrs] + obj_specs
                  + [full(a) for a in tails] + [smem]),
        out_specs=[pl.BlockSpec((1, 1), lambda g: (0, 0))] * 3,
        out_shape=out_shape,
    )(*arrs, *objs, *tails, anchors)


def _transflat5d(p5, ostr):
    rows = _ROWS[2]

    def body(p_ref, o_ref):
        t = p_ref[...].reshape(rows, 85).T
        t = jnp.pad(t, ((0, 0), (0, ostr * 128 - rows)))
        o_ref[...] = t.reshape(85, ostr, 128)

    return pl.pallas_call(
        body,
        in_specs=[pl.BlockSpec(p5.shape, lambda: (0,) * 5)],
        out_specs=pl.BlockSpec((85, ostr, 128), lambda: (0, 0, 0)),
        out_shape=jax.ShapeDtypeStruct((85, ostr, 128), jnp.float32),
    )(p5)


def kernel(p0, p1, p2, anchors, bboxes, batch_idx, cls):
    p0f = p0.reshape(_ROWS[0], 85)
    p1f = p1.reshape(_ROWS[1], 85)
    p2f = p2.reshape(_ROWS[2], 85)
    batch_idx = batch_idx.astype(jnp.int32)
    cls = cls.astype(jnp.int32)

    anc_f = anchors.reshape(-1)
    bb_f = bboxes.reshape(-1)
    t2 = _transflat5d(p2, _SSTR[2])
    ps2, got2 = _sc_layer(2, t2.reshape(-1), anc_f, bb_f, batch_idx)
    t1 = _transflat(p1f, 15360, _SSTR[1])
    ps1, got1 = _sc_layer(1, t1.reshape(-1), anc_f, bb_f, batch_idx)
    t0 = _transflat(p0f, 12288, _SSTR[0])
    ps0, got0 = _sc_layer(0, t0.reshape(-1), anc_f, bb_f, batch_idx)
    ps0, ps1, ps2 = (p.reshape(_KL, _GW) for p in (ps0, ps1, ps2))
    objs = [t0, t1, t2]

    def tile3(v):
        vp = jnp.pad(v, (0, _KPA - _NT))
        return jnp.tile(vp, 3)

    bm = jnp.stack([tile3(bboxes[:, c]).reshape(12, 128) for c in range(4)])
    bwh_c = jnp.stack([tile3(bboxes[:, 2]), tile3(bboxes[:, 3])], axis=1)
    cls_c = tile3(cls).reshape(_KL, 1)

    pbs, xcs, gots = [], [], []
    for ps, got in ((ps0, got0), (ps1, got1), (ps2, got2)):
        pbs.append(ps[:, :5].T.reshape(5, 12, 128))
        xcs.append(ps[:, 5:85])
        gots.append(got.reshape(12, 128))

    lb, lc, tot = _final_losses(pbs, xcs, gots, objs, bm, bwh_c, cls_c,
                                anchors)
    return (lb.reshape(1), lc.reshape(1), tot.reshape(1))
